# baseline (speedup 1.0000x reference)
import jax
import jax.numpy as jnp
import numpy as np
from jax import lax
from jax.experimental import pallas as pl
from jax.experimental.pallas import tpu as pltpu
from jax.experimental.pallas import tpu_sc as plsc

_B, _C, _H, _W, _G = 8, 96, 224, 224, 4
_BH = _H // _G
_NC, _NS = 2, 16
_CSC = 4
_CC = 4
_MOVES_PER_W = 4


def _inv_perms() -> np.ndarray:
    rng = np.random.RandomState(0)
    perms = np.stack([rng.permutation(_G * _G) for _ in range(_B)], axis=0)
    return np.argsort(perms, axis=1)


_INV = _inv_perms()


def _packed_inv() -> tuple[list[int], list[int]]:
    lo = [int(sum(int(v) << (4 * j) for j, v in enumerate(row[:8]))) for row in _INV]
    hi = [int(sum(int(v) << (4 * j) for j, v in enumerate(row[8:]))) for row in _INV]
    return lo, hi


_PACKED_LO, _PACKED_HI = _packed_inv()


def _sc_body(x_hbm, out_hbm, bufs, rsem, wsem):
    wid = lax.axis_index("s") * _NC + lax.axis_index("c")
    b = wid >> 2

    lo = jnp.uint32(_PACKED_LO[0])
    hi = jnp.uint32(_PACKED_HI[0])
    for bb in range(1, _B):
        lo = jnp.where(b == bb, jnp.uint32(_PACKED_LO[bb]), lo)
        hi = jnp.where(b == bb, jnp.uint32(_PACKED_HI[bb]), hi)

    def coords(k):
        t = (wid & 3) * 4 + k
        sl = (4 * jnp.where(t < 8, t, 0)).astype(jnp.uint32)
        sh_ = (4 * jnp.where(t < 8, 0, t - 8)).astype(jnp.uint32)
        src = jnp.where(t < 8, lo >> sl, hi >> sh_).astype(jnp.int32) & 15
        return src >> 2, src & 3, t >> 2, t & 3

    def read_start(k):
        sh, sw, _, _ = coords(k)
        pltpu.make_async_copy(
            x_hbm.at[b, :, pl.ds(sh * _BH, _BH), pl.ds(sw * _BH, _BH)],
            bufs[k],
            rsem[k],
        ).start()

    def read_wait(k):
        pltpu.make_async_copy(
            x_hbm.at[0, :, pl.ds(0, _BH), pl.ds(0, _BH)], bufs[k], rsem[k]
        ).wait()

    def write_start(k):
        _, _, th, tw = coords(k)
        pltpu.make_async_copy(
            bufs[k],
            out_hbm.at[b, :, pl.ds(th * _BH, _BH), pl.ds(tw * _BH, _BH)],
            wsem[k],
        ).start()

    def write_wait(k):
        pltpu.make_async_copy(
            bufs[k], out_hbm.at[0, :, pl.ds(0, _BH), pl.ds(0, _BH)], wsem[k]
        ).wait()

    for k in range(_MOVES_PER_W):
        read_start(k)
    for k in range(_MOVES_PER_W):
        read_wait(k)
        write_start(k)
    for k in range(_MOVES_PER_W):
        write_wait(k)


_sc_call = pl.kernel(
    _sc_body,
    out_type=jax.ShapeDtypeStruct((_B, _CSC, _H, _W), jnp.float32),
    mesh=plsc.VectorSubcoreMesh(core_axis_name="c", subcore_axis_name="s"),
    scratch_types=[
        [pltpu.VMEM((_CSC, _BH, _BH), jnp.float32) for _ in range(_MOVES_PER_W)],
        [pltpu.SemaphoreType.DMA for _ in range(_MOVES_PER_W)],
        [pltpu.SemaphoreType.DMA for _ in range(_MOVES_PER_W)],
    ],
    compiler_params=pltpu.CompilerParams(use_tc_tiling_on_sc=False),
    name="sc_block_permute",
)


def _tc_body(x_ref, o_ref):
    b = pl.program_id(0)
    for bb in range(_B):

        @pl.when(b == bb)
        def _():
            for t in range(_G * _G):
                src = int(_INV[bb][t])
                sh, sw = src >> 2, src & 3
                th, tw = t >> 2, t & 3
                o_ref[
                    0, :, th * _BH : (th + 1) * _BH, tw * _BH : (tw + 1) * _BH
                ] = x_ref[
                    0, :, sh * _BH : (sh + 1) * _BH, sw * _BH : (sw + 1) * _BH
                ]


def _tc_call(x):
    grid = (_B, (_C - _CSC) // _CC)
    spec = pl.BlockSpec((1, _CC, _H, _W), lambda b, c: (b, c + _CSC // _CC, 0, 0))
    return pl.pallas_call(
        _tc_body,
        grid=grid,
        in_specs=[spec],
        out_specs=spec,
        out_shape=jax.ShapeDtypeStruct((_B, _C, _H, _W), jnp.float32),
    )(x)


def kernel(x):
    xs = lax.slice_in_dim(x, 0, _CSC, axis=1)
    ps = _sc_call(xs)
    full = _tc_call(x)
    return lax.dynamic_update_slice(full, ps, (0, 0, 0, 0))


# speedup vs baseline: 1.1173x; 1.1173x over previous
import jax
import jax.numpy as jnp
import numpy as np
from jax import lax
from jax.experimental import pallas as pl
from jax.experimental.pallas import tpu as pltpu
from jax.experimental.pallas import tpu_sc as plsc

_B, _C, _H, _W, _G = 8, 96, 224, 224, 4
_BH = _H // _G
_NC, _NS = 2, 16
_CSC = 8
_CC = 8
_MOVES_PER_W = 4


def _inv_perms() -> np.ndarray:
    rng = np.random.RandomState(0)
    perms = np.stack([rng.permutation(_G * _G) for _ in range(_B)], axis=0)
    return np.argsort(perms, axis=1)


_INV = _inv_perms()


def _packed_inv() -> tuple[list[int], list[int]]:
    lo = [int(sum(int(v) << (4 * j) for j, v in enumerate(row[:8]))) for row in _INV]
    hi = [int(sum(int(v) << (4 * j) for j, v in enumerate(row[8:]))) for row in _INV]
    return lo, hi


_PACKED_LO, _PACKED_HI = _packed_inv()


def _sc_body(x_hbm, out_hbm, bufs, rsem, wsem):
    wid = lax.axis_index("s") * _NC + lax.axis_index("c")
    b = wid >> 2

    lo = jnp.uint32(_PACKED_LO[0])
    hi = jnp.uint32(_PACKED_HI[0])
    for bb in range(1, _B):
        lo = jnp.where(b == bb, jnp.uint32(_PACKED_LO[bb]), lo)
        hi = jnp.where(b == bb, jnp.uint32(_PACKED_HI[bb]), hi)

    def coords(k):
        t = (wid & 3) * 4 + k
        sl = (4 * jnp.where(t < 8, t, 0)).astype(jnp.uint32)
        sh_ = (4 * jnp.where(t < 8, 0, t - 8)).astype(jnp.uint32)
        src = jnp.where(t < 8, lo >> sl, hi >> sh_).astype(jnp.int32) & 15
        return src >> 2, src & 3, t >> 2, t & 3

    def read_start(k):
        sh, sw, _, _ = coords(k)
        pltpu.make_async_copy(
            x_hbm.at[b, :, pl.ds(sh * _BH, _BH), pl.ds(sw * _BH, _BH)],
            bufs[k],
            rsem[k],
        ).start()

    def read_wait(k):
        pltpu.make_async_copy(
            x_hbm.at[0, :, pl.ds(0, _BH), pl.ds(0, _BH)], bufs[k], rsem[k]
        ).wait()

    def write_start(k):
        _, _, th, tw = coords(k)
        pltpu.make_async_copy(
            bufs[k],
            out_hbm.at[b, :, pl.ds(th * _BH, _BH), pl.ds(tw * _BH, _BH)],
            wsem[k],
        ).start()

    def write_wait(k):
        pltpu.make_async_copy(
            bufs[k], out_hbm.at[0, :, pl.ds(0, _BH), pl.ds(0, _BH)], wsem[k]
        ).wait()

    for k in range(_MOVES_PER_W):
        read_start(k)
    for k in range(_MOVES_PER_W):
        read_wait(k)
        write_start(k)
    for k in range(_MOVES_PER_W):
        write_wait(k)


_sc_call = pl.kernel(
    _sc_body,
    out_type=jax.ShapeDtypeStruct((_B, _CSC, _H, _W), jnp.float32),
    mesh=plsc.VectorSubcoreMesh(core_axis_name="c", subcore_axis_name="s"),
    scratch_types=[
        [pltpu.VMEM((_CSC, _BH, _BH), jnp.float32) for _ in range(_MOVES_PER_W)],
        [pltpu.SemaphoreType.DMA for _ in range(_MOVES_PER_W)],
        [pltpu.SemaphoreType.DMA for _ in range(_MOVES_PER_W)],
    ],
    compiler_params=pltpu.CompilerParams(use_tc_tiling_on_sc=False),
    name="sc_block_permute",
)


def _tc_body(x_ref, o_ref):
    b = pl.program_id(0)
    for bb in range(_B):

        @pl.when(b == bb)
        def _():
            for t in range(_G * _G):
                src = int(_INV[bb][t])
                sh, sw = src >> 2, src & 3
                th, tw = t >> 2, t & 3
                o_ref[
                    0, :, th * _BH : (th + 1) * _BH, tw * _BH : (tw + 1) * _BH
                ] = x_ref[
                    0, :, sh * _BH : (sh + 1) * _BH, sw * _BH : (sw + 1) * _BH
                ]


def _tc_call(x):
    grid = (_B, (_C - _CSC) // _CC)
    spec = pl.BlockSpec((1, _CC, _H, _W), lambda b, c: (b, c + _CSC // _CC, 0, 0))
    return pl.pallas_call(
        _tc_body,
        grid=grid,
        in_specs=[spec],
        out_specs=spec,
        out_shape=jax.ShapeDtypeStruct((_B, _C, _H, _W), jnp.float32),
    )(x)


def kernel(x):
    xs = lax.slice_in_dim(x, 0, _CSC, axis=1)
    ps = _sc_call(xs)
    full = _tc_call(x)
    return lax.dynamic_update_slice(full, ps, (0, 0, 0, 0))
